# Initial kernel scaffold; baseline (speedup 1.0000x reference)
import jax
import jax.numpy as jnp
from jax.experimental import pallas as pl


def kernel(x, edge_index, batch, lin_w1, lin_b1, msg_w1, lin_w2, lin_b2, msg_w2, lin_w3, lin_b3, msg_w3):
    raise NotImplementedError("write your pallas kernel here")


# speedup vs baseline: 10.0115x; 10.0115x over previous
import functools

import jax
import jax.numpy as jnp
from jax import lax
from jax.experimental import pallas as pl
from jax.experimental.pallas import tpu as pltpu
from jax.experimental.pallas import tpu_sc as plsc

CH = 128
NB = 2


def _tile_rows(N, NS):
    rz = ((N + NS - 1) // NS + 15) // 16 * 16
    return rz


@functools.lru_cache(maxsize=None)
def _make_deg_kernel(N, E, NC, NS):
    NW = NC * NS
    NCH = -(-E // (CH * NW))
    NP = N + 16
    RZ = _tile_rows(N, NS)
    last = N - RZ * (NS - 1)
    mesh = plsc.VectorSubcoreMesh(core_axis_name="c", subcore_axis_name="s",
                                  num_cores=NC, num_subcores=NS)

    @functools.partial(
        pl.kernel,
        out_type=(
            jax.ShapeDtypeStruct((NC * N,), jnp.float32),
            jax.ShapeDtypeStruct((NC * N,), jnp.float32),
        ),
        mesh=mesh,
        scratch_types=[
            pltpu.VMEM((NCH, CH), jnp.int32),
            pltpu.VMEM((NCH, CH), jnp.int32),
            pltpu.VMEM((CH,), jnp.float32),
            pltpu.VMEM((RZ,), jnp.float32),
            pltpu.VMEM((RZ,), jnp.float32),
            pltpu.VMEM((RZ,), jnp.float32),
            pltpu.VMEM_SHARED((NP,), jnp.float32),
            pltpu.VMEM_SHARED((NP,), jnp.float32),
            pltpu.SemaphoreType.DMA,
        ],
    )
    def deg_kernel(rowd_hbm, col_hbm, odeg_out, ideg_out,
                   ridx2, cidx2, ones_v, zv, ostg, istg, oacc, iacc, sem):
        c = lax.axis_index("c")
        s = lax.axis_index("s")
        wid = s * NC + c
        r0 = s * RZ

        pltpu.sync_copy(rowd_hbm.at[wid], ridx2)
        pltpu.sync_copy(col_hbm.at[wid], cidx2)

        vone = jnp.ones((16,), jnp.float32)
        vzero = jnp.zeros((16,), jnp.float32)

        @pl.loop(0, CH // 16)
        def _(i):
            ones_v[pl.ds(i * 16, 16)] = vone

        @pl.loop(0, RZ // 16)
        def _(i):
            zv[pl.ds(i * 16, 16)] = vzero

        @pl.when(s < NS - 1)
        def _():
            pltpu.sync_copy(zv, oacc.at[pl.ds(r0, RZ)])
            pltpu.sync_copy(zv, iacc.at[pl.ds(r0, RZ)])

        @pl.when(s == NS - 1)
        def _():
            pltpu.sync_copy(zv.at[pl.ds(0, last)], oacc.at[pl.ds(r0, last)])
            pltpu.sync_copy(zv.at[pl.ds(0, last)], iacc.at[pl.ds(r0, last)])

        plsc.subcore_barrier()

        @pl.loop(0, NCH)
        def _(j):
            pltpu.async_copy(ones_v, oacc.at[ridx2.at[j]], sem, add=True)
            pltpu.async_copy(ones_v, iacc.at[cidx2.at[j]], sem, add=True)

        @pl.loop(0, 2 * NCH)
        def _(j):
            pltpu.make_async_copy(col_hbm.at[0, 0], ridx2.at[0], sem).wait()

        plsc.subcore_barrier()

        ob = pl.multiple_of(c * N + r0, 8)

        @pl.when(s < NS - 1)
        def _():
            pltpu.sync_copy(oacc.at[pl.ds(r0, RZ)], ostg)
            pltpu.sync_copy(iacc.at[pl.ds(r0, RZ)], istg)
            pltpu.sync_copy(ostg, odeg_out.at[pl.ds(ob, RZ)])
            pltpu.sync_copy(istg, ideg_out.at[pl.ds(ob, RZ)])

        @pl.when(s == NS - 1)
        def _():
            pltpu.sync_copy(oacc.at[pl.ds(r0, last)], ostg.at[pl.ds(0, last)])
            pltpu.sync_copy(iacc.at[pl.ds(r0, last)], istg.at[pl.ds(0, last)])
            pltpu.sync_copy(ostg.at[pl.ds(0, last)], odeg_out.at[pl.ds(ob, last)])
            pltpu.sync_copy(istg.at[pl.ds(0, last)], ideg_out.at[pl.ds(ob, last)])

    return deg_kernel


@functools.lru_cache(maxsize=None)
def _make_scatter_kernel(N, E, D, NC, NS):
    CHG = 64
    NW = NC * NS
    NCH = (-(-E // (CH * NW))) * (CH // CHG)
    NP = N + 16
    RZ = _tile_rows(N, NS)
    last = N - RZ * (NS - 1)
    mesh = plsc.VectorSubcoreMesh(core_axis_name="c", subcore_axis_name="s",
                                  num_cores=NC, num_subcores=NS)
    ZF = RZ // CHG
    ZFL = last // CHG
    ZREM = last - ZFL * CHG

    NI = 4

    @functools.partial(
        pl.kernel,
        out_type=jax.ShapeDtypeStruct((NC, N, D), jnp.float32),
        mesh=mesh,
        scratch_types=(
            [pltpu.VMEM_SHARED((NP, D), jnp.float32)]
            + [pltpu.VMEM((CHG,), jnp.int32)] * NI
            + [pltpu.VMEM((CHG,), jnp.int32)] * NI
            + [pltpu.VMEM((CHG, D), jnp.float32)] * NB
            + [pltpu.SemaphoreType.DMA] * NI
            + [pltpu.SemaphoreType.DMA] * NB
        ),
    )
    def scatter_kernel(xs_hbm, rowg_hbm, col_hbm, out_hbm, acc, *scr):
        ridx = scr[:NI]
        cidx = scr[NI:2 * NI]
        rows = scr[2 * NI:2 * NI + NB]
        isem = scr[2 * NI + NB:2 * NI + NB + NI]
        gsem = scr[2 * NI + NB + NI:]
        c = lax.axis_index("c")
        s = lax.axis_index("s")
        wid = s * NC + c
        r0 = s * RZ

        def idx_load(j, i):
            pltpu.async_copy(rowg_hbm.at[wid, j], ridx[i], isem[i])
            pltpu.async_copy(col_hbm.at[wid, j], cidx[i], isem[i])

        def idx_wait(j, i):
            pltpu.make_async_copy(rowg_hbm.at[wid, j], ridx[i], isem[i]).wait()
            pltpu.make_async_copy(col_hbm.at[wid, j], cidx[i], isem[i]).wait()

        vzero = jnp.zeros((16,), jnp.float32)

        @pl.loop(0, CHG)
        def _(i):
            for k in range(D // 16):
                rows[0][i, pl.ds(k * 16, 16)] = vzero

        nz = jnp.where(s < NS - 1, ZF, ZFL)

        @pl.loop(0, nz)
        def _(k):
            pltpu.sync_copy(rows[0], acc.at[pl.ds(r0 + k * CHG, CHG)])

        if ZREM:
            @pl.when(s == NS - 1)
            def _():
                pltpu.sync_copy(rows[0].at[pl.ds(0, ZREM)],
                                acc.at[pl.ds(r0 + ZFL * CHG, ZREM)])

        plsc.subcore_barrier()

        for j in range(min(NI, NCH)):
            idx_load(j, j % NI)
        for j in range(min(NB, NCH)):
            idx_wait(j, j % NI)
            pltpu.async_copy(xs_hbm.at[ridx[j % NI]], rows[j % NB],
                             gsem[j % NB])

        NG = -(-NCH // NI)

        @pl.loop(0, NG)
        def _(g):
            for u in range(NI):
                j = g * NI + u
                b = u % NB
                i = u % NI

                @pl.when(j < NCH)
                def _():
                    pltpu.make_async_copy(
                        xs_hbm.at[ridx[i]], rows[b], gsem[b]).wait()
                    pltpu.sync_copy(rows[b], acc.at[cidx[i]], add=True)

                    @pl.when(j + NI < NCH)
                    def _():
                        idx_load(j + NI, i)

                    @pl.when(j + NB < NCH)
                    def _():
                        i2 = (u + NB) % NI
                        idx_wait(j + NB, i2)
                        pltpu.async_copy(xs_hbm.at[ridx[i2]], rows[b],
                                        gsem[b])

        plsc.subcore_barrier()

        @pl.loop(0, nz)
        def _(k):
            pltpu.sync_copy(acc.at[pl.ds(r0 + k * CHG, CHG)], rows[0])
            pltpu.sync_copy(rows[0], out_hbm.at[c, pl.ds(r0 + k * CHG, CHG)])

        if ZREM:
            @pl.when(s == NS - 1)
            def _():
                pltpu.sync_copy(acc.at[pl.ds(r0 + ZFL * CHG, ZREM)],
                                rows[0].at[pl.ds(0, ZREM)])
                pltpu.sync_copy(rows[0].at[pl.ds(0, ZREM)],
                                out_hbm.at[c, pl.ds(r0 + ZFL * CHG, ZREM)])

    return scatter_kernel


def _prep_body(o0_ref, o1_ref, i0_ref, i1_ref, x_ref, dout_ref, din_ref,
               xs_ref):
    odeg = o0_ref[...] + o1_ref[...]
    ideg = i0_ref[...] + i1_ref[...]
    dout = jnp.where(odeg > 0, lax.rsqrt(odeg), 0.0)
    din = jnp.where(ideg > 0, lax.rsqrt(ideg), 0.0)
    dout_ref[...] = dout
    din_ref[...] = din
    xs_ref[...] = x_ref[...] * dout


@functools.lru_cache(maxsize=None)
def _make_prep(N, D, NC, R=1000):
    grid = N // R
    return pl.pallas_call(
        _prep_body,
        grid=(grid,),
        in_specs=[
            pl.BlockSpec((R, 1), lambda i: (i, 0)),
            pl.BlockSpec((R, 1), lambda i: (i, 0)),
            pl.BlockSpec((R, 1), lambda i: (i, 0)),
            pl.BlockSpec((R, 1), lambda i: (i, 0)),
            pl.BlockSpec((R, D), lambda i: (i, 0)),
        ],
        out_specs=[
            pl.BlockSpec((R, 1), lambda i: (i, 0)),
            pl.BlockSpec((R, 1), lambda i: (i, 0)),
            pl.BlockSpec((R, D), lambda i: (i, 0)),
        ],
        out_shape=[
            jax.ShapeDtypeStruct((N, 1), jnp.float32),
            jax.ShapeDtypeStruct((N, 1), jnp.float32),
            jax.ShapeDtypeStruct((N, D), jnp.float32),
        ],
    )


def _layer_body(act, g_ref, x_ref, din_ref, dout_ref, mw_ref, lw_ref, b_ref,
                h_ref, xs_ref):
    g = (g_ref[0] + g_ref[1]) * din_ref[...]
    acc = jnp.dot(g, mw_ref[...], preferred_element_type=jnp.float32)
    acc += jnp.dot(x_ref[...], lw_ref[...], preferred_element_type=jnp.float32)
    acc += b_ref[...]
    if act == "relu":
        h = jnp.maximum(acc, 0.0)
    else:
        h = jax.nn.sigmoid(acc)
    h_ref[...] = h
    xs_ref[...] = h * dout_ref[...]


@functools.lru_cache(maxsize=None)
def _make_layer(N, D, NC, act, R=1000):
    grid = N // R
    return pl.pallas_call(
        functools.partial(_layer_body, act),
        grid=(grid,),
        in_specs=[
            pl.BlockSpec((NC, R, D), lambda i: (0, i, 0)),
            pl.BlockSpec((R, D), lambda i: (i, 0)),
            pl.BlockSpec((R, 1), lambda i: (i, 0)),
            pl.BlockSpec((R, 1), lambda i: (i, 0)),
            pl.BlockSpec((D, D), lambda i: (0, 0)),
            pl.BlockSpec((D, D), lambda i: (0, 0)),
            pl.BlockSpec((1, D), lambda i: (0, 0)),
        ],
        out_specs=[
            pl.BlockSpec((R, D), lambda i: (i, 0)),
            pl.BlockSpec((R, D), lambda i: (i, 0)),
        ],
        out_shape=[
            jax.ShapeDtypeStruct((N, D), jnp.float32),
            jax.ShapeDtypeStruct((N, D), jnp.float32),
        ],
    )


def kernel(x, edge_index, batch, lin_w1, lin_b1, msg_w1, lin_w2, lin_b2,
           msg_w2, lin_w3, lin_b3, msg_w3):
    N, D = x.shape
    E = edge_index.shape[1]
    NC, NS = 2, 16
    NW = NC * NS
    NCH = -(-E // (CH * NW))
    EP = NW * NCH * CH
    pad = EP - E

    row = edge_index[0]
    col = edge_index[1]
    rowg = jnp.concatenate(
        [row, jnp.zeros((pad,), jnp.int32)]).reshape(NW, NCH, CH)
    rowd = jnp.concatenate(
        [row, jnp.full((pad,), N, jnp.int32)]).reshape(NW, NCH, CH)
    colp = jnp.concatenate(
        [col, jnp.full((pad,), N, jnp.int32)]).reshape(NW, NCH, CH)

    deg_k = _make_deg_kernel(N, E, NC, NS)
    odeg, ideg = deg_k(rowd, colp)

    prep = _make_prep(N, D, NC)
    dout_inv, din_inv, xs = prep(
        odeg[:N].reshape(N, 1), odeg[N:].reshape(N, 1),
        ideg[:N].reshape(N, 1), ideg[N:].reshape(N, 1), x)

    scatter = _make_scatter_kernel(N, E, D, NC, NS)
    rowg64 = rowg.reshape(NW, -1, 64)
    colp64 = colp.reshape(NW, -1, 64)
    h = x
    for lw, lb, mw, act in (
        (lin_w1, lin_b1, msg_w1, "relu"),
        (lin_w2, lin_b2, msg_w2, "relu"),
        (lin_w3, lin_b3, msg_w3, "sigmoid"),
    ):
        g = scatter(xs, rowg64, colp64)
        layer = _make_layer(N, D, NC, act)
        h, xs = layer(g, h, din_inv, dout_inv, mw, lw, lb.reshape(1, D))
    return h


# speedup vs baseline: 11.1384x; 1.1126x over previous
import functools

import jax
import jax.numpy as jnp
from jax import lax
from jax.experimental import pallas as pl
from jax.experimental.pallas import tpu as pltpu
from jax.experimental.pallas import tpu_sc as plsc

CH = 128
NB = 2


def _tile_rows(N, NS):
    rz = ((N + NS - 1) // NS + 15) // 16 * 16
    return rz


@functools.lru_cache(maxsize=None)
def _make_deg_kernel(N, E, NC, NS):
    NW = NC * NS
    NCH = -(-E // (CH * NW))
    NP = N + 16
    RZ = _tile_rows(N, NS)
    last = N - RZ * (NS - 1)
    mesh = plsc.VectorSubcoreMesh(core_axis_name="c", subcore_axis_name="s",
                                  num_cores=NC, num_subcores=NS)

    @functools.partial(
        pl.kernel,
        out_type=(
            jax.ShapeDtypeStruct((NC * N,), jnp.float32),
            jax.ShapeDtypeStruct((NC * N,), jnp.float32),
        ),
        mesh=mesh,
        scratch_types=[
            pltpu.VMEM((NCH, CH), jnp.int32),
            pltpu.VMEM((NCH, CH), jnp.int32),
            pltpu.VMEM((CH,), jnp.float32),
            pltpu.VMEM((RZ,), jnp.float32),
            pltpu.VMEM((RZ,), jnp.float32),
            pltpu.VMEM((RZ,), jnp.float32),
            pltpu.VMEM_SHARED((NP,), jnp.float32),
            pltpu.VMEM_SHARED((NP,), jnp.float32),
            pltpu.SemaphoreType.DMA,
        ],
    )
    def deg_kernel(rowd_hbm, col_hbm, odeg_out, ideg_out,
                   ridx2, cidx2, ones_v, zv, ostg, istg, oacc, iacc, sem):
        c = lax.axis_index("c")
        s = lax.axis_index("s")
        wid = s * NC + c
        r0 = s * RZ

        pltpu.sync_copy(rowd_hbm.at[wid], ridx2)
        pltpu.sync_copy(col_hbm.at[wid], cidx2)

        vone = jnp.ones((16,), jnp.float32)
        vzero = jnp.zeros((16,), jnp.float32)

        @pl.loop(0, CH // 16)
        def _(i):
            ones_v[pl.ds(i * 16, 16)] = vone

        @pl.loop(0, RZ // 16)
        def _(i):
            zv[pl.ds(i * 16, 16)] = vzero

        @pl.when(s < NS - 1)
        def _():
            pltpu.sync_copy(zv, oacc.at[pl.ds(r0, RZ)])
            pltpu.sync_copy(zv, iacc.at[pl.ds(r0, RZ)])

        @pl.when(s == NS - 1)
        def _():
            pltpu.sync_copy(zv.at[pl.ds(0, last)], oacc.at[pl.ds(r0, last)])
            pltpu.sync_copy(zv.at[pl.ds(0, last)], iacc.at[pl.ds(r0, last)])

        plsc.subcore_barrier()

        @pl.loop(0, NCH)
        def _(j):
            pltpu.async_copy(ones_v, oacc.at[ridx2.at[j]], sem, add=True)
            pltpu.async_copy(ones_v, iacc.at[cidx2.at[j]], sem, add=True)

        @pl.loop(0, 2 * NCH)
        def _(j):
            pltpu.make_async_copy(col_hbm.at[0, 0], ridx2.at[0], sem).wait()

        plsc.subcore_barrier()

        ob = pl.multiple_of(c * N + r0, 8)

        @pl.when(s < NS - 1)
        def _():
            pltpu.sync_copy(oacc.at[pl.ds(r0, RZ)], ostg)
            pltpu.sync_copy(iacc.at[pl.ds(r0, RZ)], istg)
            pltpu.sync_copy(ostg, odeg_out.at[pl.ds(ob, RZ)])
            pltpu.sync_copy(istg, ideg_out.at[pl.ds(ob, RZ)])

        @pl.when(s == NS - 1)
        def _():
            pltpu.sync_copy(oacc.at[pl.ds(r0, last)], ostg.at[pl.ds(0, last)])
            pltpu.sync_copy(iacc.at[pl.ds(r0, last)], istg.at[pl.ds(0, last)])
            pltpu.sync_copy(ostg.at[pl.ds(0, last)], odeg_out.at[pl.ds(ob, last)])
            pltpu.sync_copy(istg.at[pl.ds(0, last)], ideg_out.at[pl.ds(ob, last)])

    return deg_kernel


@functools.lru_cache(maxsize=None)
def _make_scatter_kernel(N, E, D, NC, NS):
    CHG = 64
    NW = NC * NS
    NCH = (-(-E // (CH * NW))) * (CH // CHG)
    NP = N + 16
    RZ = _tile_rows(N, NS)
    last = N - RZ * (NS - 1)
    mesh = plsc.VectorSubcoreMesh(core_axis_name="c", subcore_axis_name="s",
                                  num_cores=NC, num_subcores=NS)
    ZF = RZ // CHG
    ZFL = last // CHG
    ZREM = last - ZFL * CHG

    NI = 4

    @functools.partial(
        pl.kernel,
        out_type=jax.ShapeDtypeStruct((NC, N, D), jnp.float32),
        mesh=mesh,
        scratch_types=(
            [pltpu.VMEM_SHARED((NP, D), jnp.float32)]
            + [pltpu.VMEM((CHG,), jnp.int32)] * NI
            + [pltpu.VMEM((CHG,), jnp.int32)] * NI
            + [pltpu.VMEM((CHG, D), jnp.float32)] * NB
            + [pltpu.SemaphoreType.DMA] * NI
            + [pltpu.SemaphoreType.DMA] * NB
        ),
    )
    def scatter_kernel(xs_hbm, rowg_hbm, col_hbm, out_hbm, acc, *scr):
        ridx = scr[:NI]
        cidx = scr[NI:2 * NI]
        rows = scr[2 * NI:2 * NI + NB]
        isem = scr[2 * NI + NB:2 * NI + NB + NI]
        gsem = scr[2 * NI + NB + NI:]
        c = lax.axis_index("c")
        s = lax.axis_index("s")
        wid = s * NC + c
        r0 = s * RZ

        def idx_load(j, i):
            pltpu.async_copy(rowg_hbm.at[wid, j], ridx[i], isem[i])
            pltpu.async_copy(col_hbm.at[wid, j], cidx[i], isem[i])

        def idx_wait(j, i):
            pltpu.make_async_copy(rowg_hbm.at[wid, j], ridx[i], isem[i]).wait()
            pltpu.make_async_copy(col_hbm.at[wid, j], cidx[i], isem[i]).wait()

        vzero = jnp.zeros((16,), jnp.float32)

        @pl.loop(0, CHG)
        def _(i):
            for k in range(D // 16):
                rows[0][i, pl.ds(k * 16, 16)] = vzero

        nz = jnp.where(s < NS - 1, ZF, ZFL)

        @pl.loop(0, nz)
        def _(k):
            pltpu.sync_copy(rows[0], acc.at[pl.ds(r0 + k * CHG, CHG)])

        if ZREM:
            @pl.when(s == NS - 1)
            def _():
                pltpu.sync_copy(rows[0].at[pl.ds(0, ZREM)],
                                acc.at[pl.ds(r0 + ZFL * CHG, ZREM)])

        plsc.subcore_barrier()

        for j in range(min(NI, NCH)):
            idx_load(j, j % NI)
        for j in range(min(NB, NCH)):
            idx_wait(j, j % NI)
            pltpu.async_copy(xs_hbm.at[ridx[j % NI]], rows[j % NB],
                             gsem[j % NB])

        NG = -(-NCH // NI)

        @pl.loop(0, NG)
        def _(g):
            for u in range(NI):
                j = g * NI + u
                b = u % NB
                i = u % NI

                @pl.when(j < NCH)
                def _():
                    pltpu.make_async_copy(
                        xs_hbm.at[ridx[i]], rows[b], gsem[b]).wait()
                    pltpu.sync_copy(rows[b], acc.at[cidx[i]], add=True)

                    @pl.when(j + NI < NCH)
                    def _():
                        idx_load(j + NI, i)

                    @pl.when(j + NB < NCH)
                    def _():
                        i2 = (u + NB) % NI
                        idx_wait(j + NB, i2)
                        pltpu.async_copy(xs_hbm.at[ridx[i2]], rows[b],
                                        gsem[b])

        plsc.subcore_barrier()

        @pl.loop(0, nz)
        def _(k):
            pltpu.sync_copy(acc.at[pl.ds(r0 + k * CHG, CHG)], rows[0])
            pltpu.sync_copy(rows[0], out_hbm.at[c, pl.ds(r0 + k * CHG, CHG)])

        if ZREM:
            @pl.when(s == NS - 1)
            def _():
                pltpu.sync_copy(acc.at[pl.ds(r0 + ZFL * CHG, ZREM)],
                                rows[0].at[pl.ds(0, ZREM)])
                pltpu.sync_copy(rows[0].at[pl.ds(0, ZREM)],
                                out_hbm.at[c, pl.ds(r0 + ZFL * CHG, ZREM)])

    return scatter_kernel


def _prep_body(o0_ref, o1_ref, i0_ref, i1_ref, x_ref, dout_ref, din_ref,
               xs_ref):
    odeg = o0_ref[...] + o1_ref[...]
    ideg = i0_ref[...] + i1_ref[...]
    dout = jnp.where(odeg > 0, lax.rsqrt(odeg), 0.0)
    din = jnp.where(ideg > 0, lax.rsqrt(ideg), 0.0)
    dout_ref[...] = dout
    din_ref[...] = din
    xs_ref[...] = x_ref[...] * dout


@functools.lru_cache(maxsize=None)
def _make_prep(N, D, NC, R=1000):
    grid = N // R
    return pl.pallas_call(
        _prep_body,
        grid=(grid,),
        in_specs=[
            pl.BlockSpec((R, 1), lambda i: (i, 0)),
            pl.BlockSpec((R, 1), lambda i: (i, 0)),
            pl.BlockSpec((R, 1), lambda i: (i, 0)),
            pl.BlockSpec((R, 1), lambda i: (i, 0)),
            pl.BlockSpec((R, D), lambda i: (i, 0)),
        ],
        out_specs=[
            pl.BlockSpec((R, 1), lambda i: (i, 0)),
            pl.BlockSpec((R, 1), lambda i: (i, 0)),
            pl.BlockSpec((R, D), lambda i: (i, 0)),
        ],
        out_shape=[
            jax.ShapeDtypeStruct((N, 1), jnp.float32),
            jax.ShapeDtypeStruct((N, 1), jnp.float32),
            jax.ShapeDtypeStruct((N, D), jnp.float32),
        ],
    )


def _layer_body(act, g_ref, x_ref, din_ref, dout_ref, mw_ref, lw_ref, b_ref,
                h_ref, xs_ref):
    g = (g_ref[0] + g_ref[1]) * din_ref[...]
    acc = jnp.dot(g, mw_ref[...], preferred_element_type=jnp.float32)
    acc += jnp.dot(x_ref[...], lw_ref[...], preferred_element_type=jnp.float32)
    acc += b_ref[...]
    if act == "relu":
        h = jnp.maximum(acc, 0.0)
    else:
        h = jax.nn.sigmoid(acc)
    h_ref[...] = h
    xs_ref[...] = h * dout_ref[...]


@functools.lru_cache(maxsize=None)
def _make_layer(N, D, NC, act, R=1000):
    grid = N // R
    return pl.pallas_call(
        functools.partial(_layer_body, act),
        grid=(grid,),
        in_specs=[
            pl.BlockSpec((NC, R, D), lambda i: (0, i, 0)),
            pl.BlockSpec((R, D), lambda i: (i, 0)),
            pl.BlockSpec((R, 1), lambda i: (i, 0)),
            pl.BlockSpec((R, 1), lambda i: (i, 0)),
            pl.BlockSpec((D, D), lambda i: (0, 0)),
            pl.BlockSpec((D, D), lambda i: (0, 0)),
            pl.BlockSpec((1, D), lambda i: (0, 0)),
        ],
        out_specs=[
            pl.BlockSpec((R, D), lambda i: (i, 0)),
            pl.BlockSpec((R, D), lambda i: (i, 0)),
        ],
        out_shape=[
            jax.ShapeDtypeStruct((N, D), jnp.float32),
            jax.ShapeDtypeStruct((N, D), jnp.float32),
        ],
    )


def kernel(x, edge_index, batch, lin_w1, lin_b1, msg_w1, lin_w2, lin_b2,
           msg_w2, lin_w3, lin_b3, msg_w3):
    N, D = x.shape
    E = edge_index.shape[1]
    NC, NS = 2, 16
    NW = NC * NS
    NCH = -(-E // (CH * NW))
    EP = NW * NCH * CH
    pad = EP - E

    row = edge_index[0]
    col = edge_index[1]
    e_t = -(-E // NW)
    pad1 = NW * e_t - E
    slots = NCH * CH
    pad2 = slots - e_t
    trash1 = jnp.full((pad1,), N, jnp.int32)
    trash2 = jnp.broadcast_to(
        N + (jnp.arange(pad2, dtype=jnp.int32) % 16), (NW, pad2))

    def _tiled(base, pad1_fill, pad2_fill):
        a = jnp.concatenate([base, pad1_fill]).reshape(NW, e_t)
        return jnp.concatenate([a, pad2_fill], axis=1).reshape(NW, NCH, CH)

    zpad2 = jnp.zeros((NW, pad2), jnp.int32)
    rowg = _tiled(row, jnp.zeros((pad1,), jnp.int32), zpad2)
    rowd = _tiled(row, trash1, trash2)
    colp = _tiled(col, trash1, trash2)

    deg_k = _make_deg_kernel(N, E, NC, NS)
    odeg, ideg = deg_k(rowd, colp)

    prep = _make_prep(N, D, NC)
    dout_inv, din_inv, xs = prep(
        odeg[:N].reshape(N, 1), odeg[N:].reshape(N, 1),
        ideg[:N].reshape(N, 1), ideg[N:].reshape(N, 1), x)

    scatter = _make_scatter_kernel(N, E, D, NC, NS)
    rowg64 = rowg.reshape(NW, -1, 64)
    colp64 = colp.reshape(NW, -1, 64)
    h = x
    for lw, lb, mw, act in (
        (lin_w1, lin_b1, msg_w1, "relu"),
        (lin_w2, lin_b2, msg_w2, "relu"),
        (lin_w3, lin_b3, msg_w3, "sigmoid"),
    ):
        g = scatter(xs, rowg64, colp64)
        layer = _make_layer(N, D, NC, act)
        h, xs = layer(g, h, din_inv, dout_inv, mw, lw, lb.reshape(1, D))
    return h


# speedup vs baseline: 11.9826x; 1.0758x over previous
import functools

import jax
import jax.numpy as jnp
from jax import lax
from jax.experimental import pallas as pl
from jax.experimental.pallas import tpu as pltpu
from jax.experimental.pallas import tpu_sc as plsc

CH = 128
NB = 2


def _tile_rows(N, NS):
    rz = ((N + NS - 1) // NS + 15) // 16 * 16
    return rz


@functools.lru_cache(maxsize=None)
def _make_deg_kernel(N, E, NC, NS):
    NW = NC * NS
    NCH = -(-E // (CH * NW))
    NP = N + 16
    RZ = _tile_rows(N, NS)
    last = N - RZ * (NS - 1)
    mesh = plsc.VectorSubcoreMesh(core_axis_name="c", subcore_axis_name="s",
                                  num_cores=NC, num_subcores=NS)

    @functools.partial(
        pl.kernel,
        out_type=(
            jax.ShapeDtypeStruct((NC * N,), jnp.float32),
            jax.ShapeDtypeStruct((NC * N,), jnp.float32),
        ),
        mesh=mesh,
        scratch_types=[
            pltpu.VMEM((NCH, CH), jnp.int32),
            pltpu.VMEM((NCH, CH), jnp.int32),
            pltpu.VMEM((CH,), jnp.float32),
            pltpu.VMEM((RZ,), jnp.float32),
            pltpu.VMEM((RZ,), jnp.float32),
            pltpu.VMEM((RZ,), jnp.float32),
            pltpu.VMEM_SHARED((NP,), jnp.float32),
            pltpu.VMEM_SHARED((NP,), jnp.float32),
            pltpu.SemaphoreType.DMA,
        ],
    )
    def deg_kernel(rowd_hbm, col_hbm, odeg_out, ideg_out,
                   ridx2, cidx2, ones_v, zv, ostg, istg, oacc, iacc, sem):
        c = lax.axis_index("c")
        s = lax.axis_index("s")
        wid = s * NC + c
        r0 = s * RZ

        pltpu.sync_copy(rowd_hbm.at[wid], ridx2)
        pltpu.sync_copy(col_hbm.at[wid], cidx2)

        vone = jnp.ones((16,), jnp.float32)
        vzero = jnp.zeros((16,), jnp.float32)

        @pl.loop(0, CH // 16)
        def _(i):
            ones_v[pl.ds(i * 16, 16)] = vone

        @pl.loop(0, RZ // 16)
        def _(i):
            zv[pl.ds(i * 16, 16)] = vzero

        @pl.when(s < NS - 1)
        def _():
            pltpu.sync_copy(zv, oacc.at[pl.ds(r0, RZ)])
            pltpu.sync_copy(zv, iacc.at[pl.ds(r0, RZ)])

        @pl.when(s == NS - 1)
        def _():
            pltpu.sync_copy(zv.at[pl.ds(0, last)], oacc.at[pl.ds(r0, last)])
            pltpu.sync_copy(zv.at[pl.ds(0, last)], iacc.at[pl.ds(r0, last)])

        plsc.subcore_barrier()

        @pl.loop(0, NCH)
        def _(j):
            pltpu.async_copy(ones_v, oacc.at[ridx2.at[j]], sem, add=True)
            pltpu.async_copy(ones_v, iacc.at[cidx2.at[j]], sem, add=True)

        @pl.loop(0, 2 * NCH)
        def _(j):
            pltpu.make_async_copy(col_hbm.at[0, 0], ridx2.at[0], sem).wait()

        plsc.subcore_barrier()

        ob = pl.multiple_of(c * N + r0, 8)

        @pl.when(s < NS - 1)
        def _():
            pltpu.sync_copy(oacc.at[pl.ds(r0, RZ)], ostg)
            pltpu.sync_copy(iacc.at[pl.ds(r0, RZ)], istg)
            pltpu.sync_copy(ostg, odeg_out.at[pl.ds(ob, RZ)])
            pltpu.sync_copy(istg, ideg_out.at[pl.ds(ob, RZ)])

        @pl.when(s == NS - 1)
        def _():
            pltpu.sync_copy(oacc.at[pl.ds(r0, last)], ostg.at[pl.ds(0, last)])
            pltpu.sync_copy(iacc.at[pl.ds(r0, last)], istg.at[pl.ds(0, last)])
            pltpu.sync_copy(ostg.at[pl.ds(0, last)], odeg_out.at[pl.ds(ob, last)])
            pltpu.sync_copy(istg.at[pl.ds(0, last)], ideg_out.at[pl.ds(ob, last)])

    return deg_kernel


@functools.lru_cache(maxsize=None)
def _make_scatter_kernel(N, E, D, NC, NS):
    CHG = 128
    NW = NC * NS
    NCH = (-(-E // (CH * NW))) * (CH // CHG)
    NP = N + 16
    RZ = _tile_rows(N, NS)
    last = N - RZ * (NS - 1)
    mesh = plsc.VectorSubcoreMesh(core_axis_name="c", subcore_axis_name="s",
                                  num_cores=NC, num_subcores=NS)
    ZF = RZ // CHG
    ZFL = last // CHG
    ZREM = last - ZFL * CHG

    NI = 4

    @functools.partial(
        pl.kernel,
        out_type=jax.ShapeDtypeStruct((NC, N, D), jnp.float32),
        mesh=mesh,
        compiler_params=pltpu.CompilerParams(use_tc_tiling_on_sc=True),
        scratch_types=(
            [pltpu.VMEM_SHARED((NP, D), jnp.float32)]
            + [pltpu.VMEM((CHG,), jnp.int32)] * NI
            + [pltpu.VMEM((CHG,), jnp.int32)] * NI
            + [pltpu.VMEM((CHG, D), jnp.float32)] * NB
            + [pltpu.SemaphoreType.DMA] * NI
            + [pltpu.SemaphoreType.DMA] * NB
        ),
    )
    def scatter_kernel(xs_hbm, rowg_hbm, col_hbm, out_hbm, acc, *scr):
        ridx = scr[:NI]
        cidx = scr[NI:2 * NI]
        rows = scr[2 * NI:2 * NI + NB]
        isem = scr[2 * NI + NB:2 * NI + NB + NI]
        gsem = scr[2 * NI + NB + NI:]
        c = lax.axis_index("c")
        s = lax.axis_index("s")
        wid = s * NC + c
        r0 = s * RZ

        def idx_load(j, i):
            pltpu.async_copy(rowg_hbm.at[wid, j], ridx[i], isem[i])
            pltpu.async_copy(col_hbm.at[wid, j], cidx[i], isem[i])

        def idx_wait(j, i):
            pltpu.make_async_copy(rowg_hbm.at[wid, j], ridx[i], isem[i]).wait()
            pltpu.make_async_copy(col_hbm.at[wid, j], cidx[i], isem[i]).wait()

        vzero = jnp.zeros((16,), jnp.float32)

        @pl.loop(0, CHG)
        def _(i):
            for k in range(D // 16):
                rows[0][i, pl.ds(k * 16, 16)] = vzero

        nz = jnp.where(s < NS - 1, ZF, ZFL)

        @pl.loop(0, nz)
        def _(k):
            pltpu.sync_copy(rows[0], acc.at[pl.ds(r0 + k * CHG, CHG)])

        if ZREM:
            @pl.when(s == NS - 1)
            def _():
                pltpu.sync_copy(rows[0].at[pl.ds(0, ZREM)],
                                acc.at[pl.ds(r0 + ZFL * CHG, ZREM)])

        plsc.subcore_barrier()

        for j in range(min(NI, NCH)):
            idx_load(j, j % NI)
        for j in range(min(NB, NCH)):
            idx_wait(j, j % NI)
            pltpu.async_copy(xs_hbm.at[ridx[j % NI]], rows[j % NB],
                             gsem[j % NB])

        NG = -(-NCH // NI)

        @pl.loop(0, NG)
        def _(g):
            for u in range(NI):
                j = g * NI + u
                b = u % NB
                i = u % NI

                @pl.when(j < NCH)
                def _():
                    pltpu.make_async_copy(
                        xs_hbm.at[ridx[i]], rows[b], gsem[b]).wait()
                    pltpu.sync_copy(rows[b], acc.at[cidx[i]], add=True)

                    @pl.when(j + NI < NCH)
                    def _():
                        idx_load(j + NI, i)

                    @pl.when(j + NB < NCH)
                    def _():
                        i2 = (u + NB) % NI
                        idx_wait(j + NB, i2)
                        pltpu.async_copy(xs_hbm.at[ridx[i2]], rows[b],
                                        gsem[b])

        plsc.subcore_barrier()

        @pl.loop(0, nz)
        def _(k):
            pltpu.sync_copy(acc.at[pl.ds(r0 + k * CHG, CHG)], rows[0])
            pltpu.sync_copy(rows[0], out_hbm.at[c, pl.ds(r0 + k * CHG, CHG)])

        if ZREM:
            @pl.when(s == NS - 1)
            def _():
                pltpu.sync_copy(acc.at[pl.ds(r0 + ZFL * CHG, ZREM)],
                                rows[0].at[pl.ds(0, ZREM)])
                pltpu.sync_copy(rows[0].at[pl.ds(0, ZREM)],
                                out_hbm.at[c, pl.ds(r0 + ZFL * CHG, ZREM)])

    return scatter_kernel


def _prep_body(o0_ref, o1_ref, i0_ref, i1_ref, x_ref, dout_ref, din_ref,
               xs_ref):
    odeg = o0_ref[...] + o1_ref[...]
    ideg = i0_ref[...] + i1_ref[...]
    dout = jnp.where(odeg > 0, lax.rsqrt(odeg), 0.0)
    din = jnp.where(ideg > 0, lax.rsqrt(ideg), 0.0)
    dout_ref[...] = dout
    din_ref[...] = din
    xs_ref[...] = x_ref[...] * dout


@functools.lru_cache(maxsize=None)
def _make_prep(N, D, NC, R=1000):
    grid = N // R
    return pl.pallas_call(
        _prep_body,
        grid=(grid,),
        in_specs=[
            pl.BlockSpec((R, 1), lambda i: (i, 0)),
            pl.BlockSpec((R, 1), lambda i: (i, 0)),
            pl.BlockSpec((R, 1), lambda i: (i, 0)),
            pl.BlockSpec((R, 1), lambda i: (i, 0)),
            pl.BlockSpec((R, D), lambda i: (i, 0)),
        ],
        out_specs=[
            pl.BlockSpec((R, 1), lambda i: (i, 0)),
            pl.BlockSpec((R, 1), lambda i: (i, 0)),
            pl.BlockSpec((R, D), lambda i: (i, 0)),
        ],
        out_shape=[
            jax.ShapeDtypeStruct((N, 1), jnp.float32),
            jax.ShapeDtypeStruct((N, 1), jnp.float32),
            jax.ShapeDtypeStruct((N, D), jnp.float32),
        ],
    )


def _layer_body(act, g_ref, x_ref, din_ref, dout_ref, mw_ref, lw_ref, b_ref,
                h_ref, xs_ref):
    g = (g_ref[0] + g_ref[1]) * din_ref[...]
    acc = jnp.dot(g, mw_ref[...], preferred_element_type=jnp.float32)
    acc += jnp.dot(x_ref[...], lw_ref[...], preferred_element_type=jnp.float32)
    acc += b_ref[...]
    if act == "relu":
        h = jnp.maximum(acc, 0.0)
    else:
        h = jax.nn.sigmoid(acc)
    h_ref[...] = h
    xs_ref[...] = h * dout_ref[...]


@functools.lru_cache(maxsize=None)
def _make_layer(N, D, NC, act, R=1000):
    grid = N // R
    return pl.pallas_call(
        functools.partial(_layer_body, act),
        grid=(grid,),
        in_specs=[
            pl.BlockSpec((NC, R, D), lambda i: (0, i, 0)),
            pl.BlockSpec((R, D), lambda i: (i, 0)),
            pl.BlockSpec((R, 1), lambda i: (i, 0)),
            pl.BlockSpec((R, 1), lambda i: (i, 0)),
            pl.BlockSpec((D, D), lambda i: (0, 0)),
            pl.BlockSpec((D, D), lambda i: (0, 0)),
            pl.BlockSpec((1, D), lambda i: (0, 0)),
        ],
        out_specs=[
            pl.BlockSpec((R, D), lambda i: (i, 0)),
            pl.BlockSpec((R, D), lambda i: (i, 0)),
        ],
        out_shape=[
            jax.ShapeDtypeStruct((N, D), jnp.float32),
            jax.ShapeDtypeStruct((N, D), jnp.float32),
        ],
    )


def kernel(x, edge_index, batch, lin_w1, lin_b1, msg_w1, lin_w2, lin_b2,
           msg_w2, lin_w3, lin_b3, msg_w3):
    N, D = x.shape
    E = edge_index.shape[1]
    NC, NS = 2, 16
    NW = NC * NS
    NCH = -(-E // (CH * NW))
    EP = NW * NCH * CH
    pad = EP - E

    row = edge_index[0]
    col = edge_index[1]
    e_t = -(-E // NW)
    pad1 = NW * e_t - E
    slots = NCH * CH
    pad2 = slots - e_t
    trash1 = jnp.full((pad1,), N, jnp.int32)
    trash2 = jnp.broadcast_to(
        N + (jnp.arange(pad2, dtype=jnp.int32) % 16), (NW, pad2))

    def _tiled(base, pad1_fill, pad2_fill):
        a = jnp.concatenate([base, pad1_fill]).reshape(NW, e_t)
        return jnp.concatenate([a, pad2_fill], axis=1).reshape(NW, NCH, CH)

    zpad2 = jnp.zeros((NW, pad2), jnp.int32)
    rowg = _tiled(row, jnp.zeros((pad1,), jnp.int32), zpad2)
    rowd = _tiled(row, trash1, trash2)
    colp = _tiled(col, trash1, trash2)

    deg_k = _make_deg_kernel(N, E, NC, NS)
    odeg, ideg = deg_k(rowd, colp)

    prep = _make_prep(N, D, NC)
    dout_inv, din_inv, xs = prep(
        odeg[:N].reshape(N, 1), odeg[N:].reshape(N, 1),
        ideg[:N].reshape(N, 1), ideg[N:].reshape(N, 1), x)

    scatter = _make_scatter_kernel(N, E, D, NC, NS)
    h = x
    for lw, lb, mw, act in (
        (lin_w1, lin_b1, msg_w1, "relu"),
        (lin_w2, lin_b2, msg_w2, "relu"),
        (lin_w3, lin_b3, msg_w3, "sigmoid"),
    ):
        g = scatter(xs, rowg, colp)
        layer = _make_layer(N, D, NC, act)
        h, xs = layer(g, h, din_inv, dout_inv, mw, lw, lb.reshape(1, D))
    return h


# speedup vs baseline: 12.4328x; 1.0376x over previous
import functools

import jax
import jax.numpy as jnp
from jax import lax
from jax.experimental import pallas as pl
from jax.experimental.pallas import tpu as pltpu
from jax.experimental.pallas import tpu_sc as plsc

CH = 128
NB = 3


def _tile_rows(N, NS):
    rz = ((N + NS - 1) // NS + 15) // 16 * 16
    return rz


@functools.lru_cache(maxsize=None)
def _make_deg_kernel(N, E, NC, NS):
    NW = NC * NS
    NCH = -(-E // (CH * NW))
    NP = N + 16
    RZ = _tile_rows(N, NS)
    last = N - RZ * (NS - 1)
    mesh = plsc.VectorSubcoreMesh(core_axis_name="c", subcore_axis_name="s",
                                  num_cores=NC, num_subcores=NS)

    @functools.partial(
        pl.kernel,
        out_type=(
            jax.ShapeDtypeStruct((NC * N,), jnp.float32),
            jax.ShapeDtypeStruct((NC * N,), jnp.float32),
        ),
        mesh=mesh,
        scratch_types=[
            pltpu.VMEM((NCH, CH), jnp.int32),
            pltpu.VMEM((NCH, CH), jnp.int32),
            pltpu.VMEM((CH,), jnp.float32),
            pltpu.VMEM((RZ,), jnp.float32),
            pltpu.VMEM((RZ,), jnp.float32),
            pltpu.VMEM((RZ,), jnp.float32),
            pltpu.VMEM_SHARED((NP,), jnp.float32),
            pltpu.VMEM_SHARED((NP,), jnp.float32),
            pltpu.SemaphoreType.DMA,
        ],
    )
    def deg_kernel(rowd_hbm, col_hbm, odeg_out, ideg_out,
                   ridx2, cidx2, ones_v, zv, ostg, istg, oacc, iacc, sem):
        c = lax.axis_index("c")
        s = lax.axis_index("s")
        wid = s * NC + c
        r0 = s * RZ

        pltpu.sync_copy(rowd_hbm.at[wid], ridx2)
        pltpu.sync_copy(col_hbm.at[wid], cidx2)

        vone = jnp.ones((16,), jnp.float32)
        vzero = jnp.zeros((16,), jnp.float32)

        @pl.loop(0, CH // 16)
        def _(i):
            ones_v[pl.ds(i * 16, 16)] = vone

        @pl.loop(0, RZ // 16)
        def _(i):
            zv[pl.ds(i * 16, 16)] = vzero

        @pl.when(s < NS - 1)
        def _():
            pltpu.sync_copy(zv, oacc.at[pl.ds(r0, RZ)])
            pltpu.sync_copy(zv, iacc.at[pl.ds(r0, RZ)])

        @pl.when(s == NS - 1)
        def _():
            pltpu.sync_copy(zv.at[pl.ds(0, last)], oacc.at[pl.ds(r0, last)])
            pltpu.sync_copy(zv.at[pl.ds(0, last)], iacc.at[pl.ds(r0, last)])

        plsc.subcore_barrier()

        @pl.loop(0, NCH)
        def _(j):
            pltpu.async_copy(ones_v, oacc.at[ridx2.at[j]], sem, add=True)
            pltpu.async_copy(ones_v, iacc.at[cidx2.at[j]], sem, add=True)

        @pl.loop(0, 2 * NCH)
        def _(j):
            pltpu.make_async_copy(col_hbm.at[0, 0], ridx2.at[0], sem).wait()

        plsc.subcore_barrier()

        ob = pl.multiple_of(c * N + r0, 8)

        @pl.when(s < NS - 1)
        def _():
            pltpu.sync_copy(oacc.at[pl.ds(r0, RZ)], ostg)
            pltpu.sync_copy(iacc.at[pl.ds(r0, RZ)], istg)
            pltpu.sync_copy(ostg, odeg_out.at[pl.ds(ob, RZ)])
            pltpu.sync_copy(istg, ideg_out.at[pl.ds(ob, RZ)])

        @pl.when(s == NS - 1)
        def _():
            pltpu.sync_copy(oacc.at[pl.ds(r0, last)], ostg.at[pl.ds(0, last)])
            pltpu.sync_copy(iacc.at[pl.ds(r0, last)], istg.at[pl.ds(0, last)])
            pltpu.sync_copy(ostg.at[pl.ds(0, last)], odeg_out.at[pl.ds(ob, last)])
            pltpu.sync_copy(istg.at[pl.ds(0, last)], ideg_out.at[pl.ds(ob, last)])

    return deg_kernel


@functools.lru_cache(maxsize=None)
def _make_scatter_kernel(N, E, D, NC, NS):
    CHG = 128
    NW = NC * NS
    NCH = (-(-E // (CH * NW))) * (CH // CHG)
    NP = N + 16
    RZ = _tile_rows(N, NS)
    last = N - RZ * (NS - 1)
    mesh = plsc.VectorSubcoreMesh(core_axis_name="c", subcore_axis_name="s",
                                  num_cores=NC, num_subcores=NS)
    ZF = RZ // CHG
    ZFL = last // CHG
    ZREM = last - ZFL * CHG

    NI = 6

    @functools.partial(
        pl.kernel,
        out_type=jax.ShapeDtypeStruct((NC, N, D), jnp.float32),
        mesh=mesh,
        compiler_params=pltpu.CompilerParams(use_tc_tiling_on_sc=True),
        scratch_types=(
            [pltpu.VMEM_SHARED((NP, D), jnp.float32)]
            + [pltpu.VMEM((CHG,), jnp.int32)] * NI
            + [pltpu.VMEM((CHG,), jnp.int32)] * NI
            + [pltpu.VMEM((CHG, D), jnp.float32)] * NB
            + [pltpu.SemaphoreType.DMA] * NI
            + [pltpu.SemaphoreType.DMA] * NB
        ),
    )
    def scatter_kernel(xs_hbm, rowg_hbm, col_hbm, out_hbm, acc, *scr):
        ridx = scr[:NI]
        cidx = scr[NI:2 * NI]
        rows = scr[2 * NI:2 * NI + NB]
        isem = scr[2 * NI + NB:2 * NI + NB + NI]
        gsem = scr[2 * NI + NB + NI:]
        c = lax.axis_index("c")
        s = lax.axis_index("s")
        wid = s * NC + c
        r0 = s * RZ

        def idx_load(j, i):
            pltpu.async_copy(rowg_hbm.at[wid, j], ridx[i], isem[i])
            pltpu.async_copy(col_hbm.at[wid, j], cidx[i], isem[i])

        def idx_wait(j, i):
            pltpu.make_async_copy(rowg_hbm.at[wid, j], ridx[i], isem[i]).wait()
            pltpu.make_async_copy(col_hbm.at[wid, j], cidx[i], isem[i]).wait()

        vzero = jnp.zeros((16,), jnp.float32)

        @pl.loop(0, CHG)
        def _(i):
            for k in range(D // 16):
                rows[0][i, pl.ds(k * 16, 16)] = vzero

        nz = jnp.where(s < NS - 1, ZF, ZFL)

        @pl.loop(0, nz)
        def _(k):
            pltpu.sync_copy(rows[0], acc.at[pl.ds(r0 + k * CHG, CHG)])

        if ZREM:
            @pl.when(s == NS - 1)
            def _():
                pltpu.sync_copy(rows[0].at[pl.ds(0, ZREM)],
                                acc.at[pl.ds(r0 + ZFL * CHG, ZREM)])

        plsc.subcore_barrier()

        for j in range(min(NI, NCH)):
            idx_load(j, j % NI)
        for j in range(min(NB, NCH)):
            idx_wait(j, j % NI)
            pltpu.async_copy(xs_hbm.at[ridx[j % NI]], rows[j % NB],
                             gsem[j % NB])

        NG = -(-NCH // NI)

        @pl.loop(0, NG)
        def _(g):
            for u in range(NI):
                j = g * NI + u
                b = u % NB
                i = u % NI

                @pl.when(j < NCH)
                def _():
                    pltpu.make_async_copy(
                        xs_hbm.at[ridx[i]], rows[b], gsem[b]).wait()
                    pltpu.sync_copy(rows[b], acc.at[cidx[i]], add=True)

                    @pl.when(j + NI < NCH)
                    def _():
                        idx_load(j + NI, i)

                    @pl.when(j + NB < NCH)
                    def _():
                        i2 = (u + NB) % NI
                        idx_wait(j + NB, i2)
                        pltpu.async_copy(xs_hbm.at[ridx[i2]], rows[b],
                                        gsem[b])

        plsc.subcore_barrier()

        @pl.loop(0, nz)
        def _(k):
            pltpu.sync_copy(acc.at[pl.ds(r0 + k * CHG, CHG)], rows[0])
            pltpu.sync_copy(rows[0], out_hbm.at[c, pl.ds(r0 + k * CHG, CHG)])

        if ZREM:
            @pl.when(s == NS - 1)
            def _():
                pltpu.sync_copy(acc.at[pl.ds(r0 + ZFL * CHG, ZREM)],
                                rows[0].at[pl.ds(0, ZREM)])
                pltpu.sync_copy(rows[0].at[pl.ds(0, ZREM)],
                                out_hbm.at[c, pl.ds(r0 + ZFL * CHG, ZREM)])

    return scatter_kernel


def _prep_body(o0_ref, o1_ref, i0_ref, i1_ref, x_ref, dout_ref, din_ref,
               xs_ref):
    odeg = o0_ref[...] + o1_ref[...]
    ideg = i0_ref[...] + i1_ref[...]
    dout = jnp.where(odeg > 0, lax.rsqrt(odeg), 0.0)
    din = jnp.where(ideg > 0, lax.rsqrt(ideg), 0.0)
    dout_ref[...] = dout
    din_ref[...] = din
    xs_ref[...] = x_ref[...] * dout


@functools.lru_cache(maxsize=None)
def _make_prep(N, D, NC, R=1000):
    grid = N // R
    return pl.pallas_call(
        _prep_body,
        grid=(grid,),
        in_specs=[
            pl.BlockSpec((R, 1), lambda i: (i, 0)),
            pl.BlockSpec((R, 1), lambda i: (i, 0)),
            pl.BlockSpec((R, 1), lambda i: (i, 0)),
            pl.BlockSpec((R, 1), lambda i: (i, 0)),
            pl.BlockSpec((R, D), lambda i: (i, 0)),
        ],
        out_specs=[
            pl.BlockSpec((R, 1), lambda i: (i, 0)),
            pl.BlockSpec((R, 1), lambda i: (i, 0)),
            pl.BlockSpec((R, D), lambda i: (i, 0)),
        ],
        out_shape=[
            jax.ShapeDtypeStruct((N, 1), jnp.float32),
            jax.ShapeDtypeStruct((N, 1), jnp.float32),
            jax.ShapeDtypeStruct((N, D), jnp.float32),
        ],
    )


def _layer_body(act, g_ref, x_ref, din_ref, dout_ref, mw_ref, lw_ref, b_ref,
                h_ref, xs_ref):
    g = (g_ref[0] + g_ref[1]) * din_ref[...]
    acc = jnp.dot(g, mw_ref[...], preferred_element_type=jnp.float32)
    acc += jnp.dot(x_ref[...], lw_ref[...], preferred_element_type=jnp.float32)
    acc += b_ref[...]
    if act == "relu":
        h = jnp.maximum(acc, 0.0)
    else:
        h = jax.nn.sigmoid(acc)
    h_ref[...] = h
    xs_ref[...] = h * dout_ref[...]


@functools.lru_cache(maxsize=None)
def _make_layer(N, D, NC, act, R=1000):
    grid = N // R
    return pl.pallas_call(
        functools.partial(_layer_body, act),
        grid=(grid,),
        in_specs=[
            pl.BlockSpec((NC, R, D), lambda i: (0, i, 0)),
            pl.BlockSpec((R, D), lambda i: (i, 0)),
            pl.BlockSpec((R, 1), lambda i: (i, 0)),
            pl.BlockSpec((R, 1), lambda i: (i, 0)),
            pl.BlockSpec((D, D), lambda i: (0, 0)),
            pl.BlockSpec((D, D), lambda i: (0, 0)),
            pl.BlockSpec((1, D), lambda i: (0, 0)),
        ],
        out_specs=[
            pl.BlockSpec((R, D), lambda i: (i, 0)),
            pl.BlockSpec((R, D), lambda i: (i, 0)),
        ],
        out_shape=[
            jax.ShapeDtypeStruct((N, D), jnp.float32),
            jax.ShapeDtypeStruct((N, D), jnp.float32),
        ],
    )


def kernel(x, edge_index, batch, lin_w1, lin_b1, msg_w1, lin_w2, lin_b2,
           msg_w2, lin_w3, lin_b3, msg_w3):
    N, D = x.shape
    E = edge_index.shape[1]
    NC, NS = 2, 16
    NW = NC * NS
    NCH = -(-E // (CH * NW))
    EP = NW * NCH * CH
    pad = EP - E

    row = edge_index[0]
    col = edge_index[1]
    e_t = -(-E // NW)
    pad1 = NW * e_t - E
    slots = NCH * CH
    pad2 = slots - e_t
    trash1 = jnp.full((pad1,), N, jnp.int32)
    trash2 = jnp.broadcast_to(
        N + (jnp.arange(pad2, dtype=jnp.int32) % 16), (NW, pad2))

    def _tiled(base, pad1_fill, pad2_fill):
        a = jnp.concatenate([base, pad1_fill]).reshape(NW, e_t)
        return jnp.concatenate([a, pad2_fill], axis=1).reshape(NW, NCH, CH)

    zpad2 = jnp.zeros((NW, pad2), jnp.int32)
    rowg = _tiled(row, jnp.zeros((pad1,), jnp.int32), zpad2)
    rowd = _tiled(row, trash1, trash2)
    colp = _tiled(col, trash1, trash2)

    deg_k = _make_deg_kernel(N, E, NC, NS)
    odeg, ideg = deg_k(rowd, colp)

    prep = _make_prep(N, D, NC)
    dout_inv, din_inv, xs = prep(
        odeg[:N].reshape(N, 1), odeg[N:].reshape(N, 1),
        ideg[:N].reshape(N, 1), ideg[N:].reshape(N, 1), x)

    scatter = _make_scatter_kernel(N, E, D, NC, NS)
    h = x
    for lw, lb, mw, act in (
        (lin_w1, lin_b1, msg_w1, "relu"),
        (lin_w2, lin_b2, msg_w2, "relu"),
        (lin_w3, lin_b3, msg_w3, "sigmoid"),
    ):
        g = scatter(xs, rowg, colp)
        layer = _make_layer(N, D, NC, act)
        h, xs = layer(g, h, din_inv, dout_inv, mw, lw, lb.reshape(1, D))
    return h
